# baseline (speedup 1.0000x reference)
import jax
import jax.numpy as jnp
from jax import lax
from jax.experimental import pallas as pl
from jax.experimental.pallas import tpu as pltpu
from jax.experimental.pallas import tpu_sc as plsc

S = 200
B = 4096
D = 64
L = 16

NC = 2
NS = 16
NW = NC * NS
BW = B // NW
NG = BW // L
N_ROWS2 = 500000


def _body(src_hbm, table_hbm, pe_hbm, out_hbm,
          idx_v, pe_v, slab0, slab1, tout0, tout1, i2_0, i2_1,
          g0, g1, w0, w1):
    wid = lax.axis_index("s") * NC + lax.axis_index("c")
    bcol = wid * BW

    pltpu.sync_copy(src_hbm.at[:, pl.ds(bcol, BW)], idx_v)
    pltpu.sync_copy(pe_hbm, pe_v)

    def prep_gather(s, i2_v, slab_v, gsem):
        for g in range(NG):
            v = idx_v[s, pl.ds(g * L, L)]
            i2_v[pl.ds(g * L, L)] = jax.lax.shift_right_logical(v, 1)
        pltpu.async_copy(table_hbm.at[i2_v], slab_v, gsem)

    def compute(s, slab_v, tout_v):
        iota = lax.iota(jnp.int32, L)
        bvecs = tuple(iota + g * L for g in range(NG))
        cbase = tuple(
            jax.lax.shift_left((idx_v[s, pl.ds(g * L, L)] & 1), 6)
            for g in range(NG)
        )

        def d_body(d, carry):
            dsplat = jnp.full((L,), 0, jnp.int32) + d
            pes = plsc.load_gather(pe_v, [dsplat * 0 + s, dsplat])
            for g in range(NG):
                vals = plsc.load_gather(slab_v, [bvecs[g], cbase[g] + dsplat])
                tout_v[d, pl.ds(g * L, L)] = vals + pes
            return carry

        lax.fori_loop(0, D, d_body, 0)

    def emit(s, tout_v, wsem):
        pltpu.async_copy(tout_v, out_hbm.at[s, :, pl.ds(bcol, BW)], wsem)

    def wait_g(sem, slab_v):
        pltpu.make_async_copy(table_hbm.at[i2_0], slab_v, sem).wait()

    def wait_w(sem, tout_v):
        pltpu.make_async_copy(tout_v, out_hbm.at[0, :, pl.ds(bcol, BW)], sem).wait()

    prep_gather(0, i2_0, slab0, g0)
    prep_gather(1, i2_1, slab1, g1)

    bufs = ((slab0, tout0, i2_0, g0, w0), (slab1, tout1, i2_1, g1, w1))

    def step(i, carry):
        for b in range(2):
            s = 2 * i + b
            slab_v, tout_v, i2_v, gsem, wsem = bufs[b]

            @pl.when(s >= 2)
            def _():
                wait_w(wsem, tout_v)

            wait_g(gsem, slab_v)
            compute(s, slab_v, tout_v)

            @pl.when(s + 2 < S)
            def _():
                prep_gather(s + 2, i2_v, slab_v, gsem)

            emit(s, tout_v, wsem)
        return carry

    lax.fori_loop(0, S // 2, step, 0)

    wait_w(w0, tout0)
    wait_w(w1, tout1)


@jax.jit
def _pe_embed(src, table2, pe2d):
    mesh = plsc.VectorSubcoreMesh(core_axis_name="c", subcore_axis_name="s")
    k = pl.kernel(
        _body,
        out_type=jax.ShapeDtypeStruct((S, D, B), jnp.float32),
        mesh=mesh,
        scratch_types=[
            pltpu.VMEM((S, BW), jnp.int32),
            pltpu.VMEM((S, D), jnp.float32),
            pltpu.VMEM((BW, 2 * D), jnp.float32),
            pltpu.VMEM((BW, 2 * D), jnp.float32),
            pltpu.VMEM((D, BW), jnp.float32),
            pltpu.VMEM((D, BW), jnp.float32),
            pltpu.VMEM((BW,), jnp.int32),
            pltpu.VMEM((BW,), jnp.int32),
            pltpu.SemaphoreType.DMA,
            pltpu.SemaphoreType.DMA,
            pltpu.SemaphoreType.DMA,
            pltpu.SemaphoreType.DMA,
        ],
        compiler_params=pltpu.CompilerParams(
            use_tc_tiling_on_sc=False, needs_layout_passes=False
        ),
    )
    return k(src, table2, pe2d)


def kernel(src, table, pe):
    src = src.astype(jnp.int32)
    table2 = table.reshape(N_ROWS2, 2 * D)
    pe2d = pe.reshape(S, D)
    out_t = _pe_embed(src, table2, pe2d)
    return out_t.transpose(0, 2, 1)


# speedup vs baseline: 1.6583x; 1.6583x over previous
import jax
import jax.numpy as jnp
from jax import lax
from jax.experimental import pallas as pl
from jax.experimental.pallas import tpu as pltpu
from jax.experimental.pallas import tpu_sc as plsc

S = 200
B = 4096
D = 64
L = 16

NC = 2
NS = 16
NW = NC * NS
BW = B // NW
NG = BW // L
N_ROWS2 = 500000


def _body(src_hbm, table_hbm, pe_hbm, out_hbm,
          idx_v, pe_v, slab0, slab1, tout0, tout1, i2_0, i2_1,
          g0, g1, w0, w1):
    wid = lax.axis_index("s") * NC + lax.axis_index("c")
    bcol = wid * BW

    pltpu.sync_copy(src_hbm.at[:, pl.ds(bcol, BW)], idx_v)
    pltpu.sync_copy(pe_hbm, pe_v)

    iota = lax.iota(jnp.int32, L)

    def prep_gather(s, i2_v, slab_v, gsem):
        for g in range(NG):
            v = idx_v[s, pl.ds(g * L, L)]
            i2_v[pl.ds(g * L, L)] = jax.lax.shift_right_logical(v, 1)
        pltpu.async_copy(table_hbm.at[i2_v], slab_v, gsem)

    def compute(s, slab_v, tout_v):
        pe16 = tuple(pe_v[s, pl.ds(dg * L, L)] for dg in range(4))
        half64 = tuple(
            jax.lax.shift_left(idx_v[s, pl.ds(g * L, L)] & 1, 6)
            for g in range(NG)
        )
        bconst = tuple(g * L + iota for g in range(NG))

        def r_body(r, carry):
            rot = (iota + r) & 15
            for dg in range(4):
                crot = dg * L + rot
                perot = pe16[dg].at[rot].get(mode="promise_in_bounds")
                for g in range(NG):
                    vals = plsc.load_gather(
                        slab_v, [bconst[g], half64[g] + crot]) + perot
                    plsc.store_scatter(tout_v, [crot, bconst[g]], vals)
            return carry

        lax.fori_loop(0, L, r_body, 0)

    def emit(s, tout_v, wsem):
        pltpu.async_copy(tout_v, out_hbm.at[s, :, pl.ds(bcol, BW)], wsem)

    def wait_g(sem, slab_v):
        pltpu.make_async_copy(table_hbm.at[i2_0], slab_v, sem).wait()

    def wait_w(sem, tout_v):
        pltpu.make_async_copy(tout_v, out_hbm.at[0, :, pl.ds(bcol, BW)],
                              sem).wait()

    prep_gather(0, i2_0, slab0, g0)
    prep_gather(1, i2_1, slab1, g1)

    bufs = ((slab0, tout0, i2_0, g0, w0), (slab1, tout1, i2_1, g1, w1))

    def step(i, carry):
        for b in range(2):
            s = 2 * i + b
            slab_v, tout_v, i2_v, gsem, wsem = bufs[b]

            @pl.when(s >= 2)
            def _():
                wait_w(wsem, tout_v)

            wait_g(gsem, slab_v)
            compute(s, slab_v, tout_v)

            @pl.when(s + 2 < S)
            def _():
                prep_gather(s + 2, i2_v, slab_v, gsem)

            emit(s, tout_v, wsem)
        return carry

    lax.fori_loop(0, S // 2, step, 0)

    wait_w(w0, tout0)
    wait_w(w1, tout1)


@jax.jit
def _pe_embed(src, table2, pe2d):
    mesh = plsc.VectorSubcoreMesh(core_axis_name="c", subcore_axis_name="s")
    k = pl.kernel(
        _body,
        out_type=jax.ShapeDtypeStruct((S, D, B), jnp.float32),
        mesh=mesh,
        scratch_types=[
            pltpu.VMEM((S, BW), jnp.int32),
            pltpu.VMEM((S, D), jnp.float32),
            pltpu.VMEM((BW, 2 * D), jnp.float32),
            pltpu.VMEM((BW, 2 * D), jnp.float32),
            pltpu.VMEM((D, BW), jnp.float32),
            pltpu.VMEM((D, BW), jnp.float32),
            pltpu.VMEM((BW,), jnp.int32),
            pltpu.VMEM((BW,), jnp.int32),
            pltpu.SemaphoreType.DMA,
            pltpu.SemaphoreType.DMA,
            pltpu.SemaphoreType.DMA,
            pltpu.SemaphoreType.DMA,
        ],
        compiler_params=pltpu.CompilerParams(
            use_tc_tiling_on_sc=False, needs_layout_passes=False
        ),
    )
    return k(src, table2, pe2d)


def kernel(src, table, pe):
    src = src.astype(jnp.int32)
    table2 = table.reshape(N_ROWS2, 2 * D)
    pe2d = pe.reshape(S, D)
    out_t = _pe_embed(src, table2, pe2d)
    return out_t.transpose(0, 2, 1)


# speedup vs baseline: 1.8976x; 1.1443x over previous
import jax
import jax.numpy as jnp
from jax import lax
from jax.experimental import pallas as pl
from jax.experimental.pallas import tpu as pltpu
from jax.experimental.pallas import tpu_sc as plsc

S = 200
B = 4096
D = 64
L = 16

NC = 2
NS = 16
NW = NC * NS
BW = B // NW
NG = BW // L
N_ROWS2 = 500000


def _body(src_hbm, table_hbm, pe_hbm, out_hbm,
          idx_v, pe_v, slab0, slab1, tout0, tout1, i2_0, i2_1,
          g0, g1, w0, w1):
    wid = lax.axis_index("s") * NC + lax.axis_index("c")
    bcol = wid * BW

    pltpu.sync_copy(src_hbm.at[:, pl.ds(bcol, BW)], idx_v)
    pltpu.sync_copy(pe_hbm, pe_v)

    iota = lax.iota(jnp.int32, L)

    def prep_gather(s, i2_v, slab_v, gsem):
        for g in range(NG):
            v = idx_v[s, pl.ds(g * L, L)]
            i2_v[pl.ds(g * L, L)] = jax.lax.shift_right_logical(v, 1)
        pltpu.async_copy(table_hbm.at[i2_v], slab_v, gsem)

    def compute(s, slab_v, tout_v):
        pe16 = tuple(pe_v[s, pl.ds(dg * L, L)] for dg in range(4))
        half64 = tuple(
            jax.lax.shift_left(idx_v[s, pl.ds(g * L, L)] & 1, 6)
            for g in range(NG)
        )
        bconst = tuple(g * L + iota for g in range(NG))

        def r_body(r, carry):
            rot = (iota + r) & 15
            for dg in range(4):
                crot = dg * L + rot
                perot = pe16[dg].at[rot].get(mode="promise_in_bounds")
                for g in range(NG):
                    vals = plsc.load_gather(
                        slab_v, [bconst[g], half64[g] + crot]) + perot
                    plsc.store_scatter(tout_v, [crot, bconst[g]], vals)
            return carry

        lax.fori_loop(0, L, r_body, 0)

    def emit(s, tout_v, wsem):
        pltpu.async_copy(tout_v, out_hbm.at[s, :, pl.ds(bcol, BW)], wsem)

    def wait_g(sem, slab_v):
        pltpu.make_async_copy(table_hbm.at[i2_0], slab_v, sem).wait()

    def wait_w(sem, tout_v):
        pltpu.make_async_copy(tout_v, out_hbm.at[0, :, pl.ds(bcol, BW)],
                              sem).wait()

    prep_gather(0, i2_0, slab0, g0)
    prep_gather(1, i2_1, slab1, g1)

    bufs = ((slab0, tout0, i2_0, g0, w0), (slab1, tout1, i2_1, g1, w1))

    def step(i, carry):
        for b in range(2):
            s = 2 * i + b
            slab_v, tout_v, i2_v, gsem, wsem = bufs[b]

            @pl.when(s >= 2)
            def _():
                wait_w(wsem, tout_v)

            wait_g(gsem, slab_v)
            compute(s, slab_v, tout_v)

            @pl.when(s + 2 < S)
            def _():
                prep_gather(s + 2, i2_v, slab_v, gsem)

            emit(s, tout_v, wsem)
        return carry

    lax.fori_loop(0, S // 2, step, 0)

    wait_w(w0, tout0)
    wait_w(w1, tout1)


@jax.jit
def _pe_embed(src, table2, pe2d):
    mesh = plsc.VectorSubcoreMesh(core_axis_name="c", subcore_axis_name="s")
    k = pl.kernel(
        _body,
        out_type=jax.ShapeDtypeStruct((S, D, B), jnp.float32),
        mesh=mesh,
        scratch_types=[
            pltpu.VMEM((S, BW), jnp.int32),
            pltpu.VMEM((S, 2 * D), jnp.float32),
            pltpu.VMEM((BW, 2 * D), jnp.float32),
            pltpu.VMEM((BW, 2 * D), jnp.float32),
            pltpu.VMEM((D, BW), jnp.float32),
            pltpu.VMEM((D, BW), jnp.float32),
            pltpu.VMEM((BW,), jnp.int32),
            pltpu.VMEM((BW,), jnp.int32),
            pltpu.SemaphoreType.DMA,
            pltpu.SemaphoreType.DMA,
            pltpu.SemaphoreType.DMA,
            pltpu.SemaphoreType.DMA,
        ],
        compiler_params=pltpu.CompilerParams(
            use_tc_tiling_on_sc=True, needs_layout_passes=False
        ),
    )
    return k(src, table2, pe2d)


def kernel(src, table, pe):
    src = src.astype(jnp.int32)
    table2 = table.reshape(N_ROWS2, 2 * D)
    pe2d = jnp.pad(pe.reshape(S, D), ((0, 0), (0, D)))
    out_t = _pe_embed(src, table2, pe2d)
    return out_t.transpose(0, 2, 1)
